# baseline (speedup 1.0000x reference)
import functools

import jax
import jax.numpy as jnp
from jax import lax
from jax.experimental import pallas as pl
from jax.experimental.pallas import tpu as pltpu
from jax.experimental.pallas import tpu_sc as plsc

_NC = 2
_NS = 16
_NW = _NC * _NS
_CH = 128
_NBUF = 8
_L = 4


@functools.lru_cache(maxsize=None)
def _build(B, V, D):
    assert B % (_NW * _CH) == 0
    b_per_w = B // _NW
    n_ch = b_per_w // _CH
    assert n_ch % _NBUF == 0 and n_ch > 2 * _NBUF

    mesh = plsc.VectorSubcoreMesh(core_axis_name="c", subcore_axis_name="s")

    @functools.partial(
        pl.kernel,
        out_type=jax.ShapeDtypeStruct((B, D), jnp.float32),
        mesh=mesh,
        scratch_types=[
            pltpu.VMEM((n_ch, _CH), jnp.int32),
            pltpu.VMEM((_NBUF, _CH, D), jnp.float32),
            [pltpu.SemaphoreType.DMA] * _NBUF,
            [pltpu.SemaphoreType.DMA] * _NBUF,
        ],
        compiler_params=pltpu.CompilerParams(use_tc_tiling_on_sc=False),
    )
    def gather_kernel(idx_hbm, table_hbm, out_hbm, idx_v, rows_v, gsems, osems):
        wid = lax.axis_index("s") * _NC + lax.axis_index("c")
        base = wid * b_per_w
        pltpu.sync_copy(idx_hbm.at[wid], idx_v)

        def fire_gather(c, slot):
            pltpu.async_copy(table_hbm.at[idx_v.at[c]], rows_v.at[slot],
                             gsems[slot])

        def wait_gather(c, slot):
            pltpu.make_async_copy(table_hbm.at[idx_v.at[c]], rows_v.at[slot],
                                  gsems[slot]).wait()

        def fire_store(c, slot):
            pltpu.async_copy(rows_v.at[slot],
                             out_hbm.at[pl.ds(base + c * _CH, _CH)],
                             osems[slot])

        def wait_store(c, slot):
            pltpu.make_async_copy(rows_v.at[slot],
                                  out_hbm.at[pl.ds(base + c * _CH, _CH)],
                                  osems[slot]).wait()

        for c in range(_L):
            fire_gather(c, c % _NBUF)
        for c in range(_NBUF - _L):
            fire_gather(c + _L, (c + _L) % _NBUF)
            wait_gather(c, c % _NBUF)
            fire_store(c, c % _NBUF)

        @pl.loop(_NBUF - _L, n_ch - _L, step=_NBUF)
        def _steady(j):
            for b in range(_NBUF):
                c = j + b
                slot = (b + _NBUF - _L) % _NBUF
                wait_store(c + _L - _NBUF, b)
                fire_gather(c + _L, b)
                wait_gather(c, slot)
                fire_store(c, slot)

        for c in range(n_ch - _L, n_ch):
            wait_gather(c, c % _NBUF)
            fire_store(c, c % _NBUF)
        for c in range(n_ch - _NBUF, n_ch):
            wait_store(c, c % _NBUF)

    return gather_kernel


def kernel(indices_, table):
    Bb, H = indices_.shape
    V, D = table.shape
    B = Bb * H
    b_per_w = B // _NW
    idx3 = indices_.reshape(_NW, b_per_w // _CH, _CH).astype(jnp.int32)
    out = _build(B, V, D)(idx3, table)
    return out.reshape(Bb, H, D)
